# baseline (speedup 1.0000x reference)
import functools

import jax
import jax.numpy as jnp
from jax import lax
from jax.experimental import pallas as pl
from jax.experimental.pallas import tpu as pltpu
from jax.experimental.pallas import tpu_sc as plsc

N_USERS = 25000
N_ITEMS = 25000
HALF = 25000
N_NODES = N_USERS + N_ITEMS
D = 64
NUM_LAYERS = 3
E = 800000

NTILES = 16
NW = 32
ROWS_PT = 1568
PAD_HALF = NTILES * ROWS_PT
PADN = 2 * PAD_HALF
PAD_OFF = PAD_HALF - HALF

CH = 128
CPS = 8
SLAB = CH * CPS
EPAD = 802816
WCAP = EPAD // NW
N1 = EPAD + SLAB

SLABP = 784
NSLABP = WCAP // SLABP

NBUF = 3

BLK = 56
NBLK = ROWS_PT // BLK

_mesh = plsc.VectorSubcoreMesh(core_axis_name="c", subcore_axis_name="s")
_sc_params = pltpu.CompilerParams(use_tc_tiling_on_sc=False)
_sc_params_nlp = pltpu.CompilerParams(use_tc_tiling_on_sc=False,
                                      needs_layout_passes=False)


def _zeros16():
    return jnp.zeros((16,), jnp.float32)


@functools.partial(
    pl.kernel,
    mesh=_mesh,
    out_type=[
        jax.ShapeDtypeStruct((2, N1), jnp.int32),
        jax.ShapeDtypeStruct((2, N1), jnp.int32),
        jax.ShapeDtypeStruct((2, NW, 16), jnp.int32),
    ],
    scratch_types=[
        pltpu.VMEM((SLABP,), jnp.int32),
        pltpu.VMEM((SLABP,), jnp.int32),
        pltpu.VMEM((WCAP + 16,), jnp.int32),
        pltpu.VMEM((WCAP + 16,), jnp.int32),
        pltpu.VMEM((WCAP + 16,), jnp.int32),
        pltpu.VMEM((WCAP + 16,), jnp.int32),
        pltpu.VMEM((2, 16), jnp.int32),
    ],
    compiler_params=_sc_params_nlp,
)
def _part_kernel(row_hbm, col_hbm, brow_hbm, bcol_hbm, cnts_hbm,
                 rin, cin, lr, lc, hr, hc, cntb):
    sc = lax.axis_index("c")
    t = lax.axis_index("s")
    w = sc * NTILES + t
    ebase = w * WCAP
    half16 = jnp.full((16,), HALF, jnp.int32)
    shift16 = jnp.full((16,), PAD_OFF, jnp.int32)

    def slab_body(s, carry):
        cl16, ch16 = carry
        pltpu.sync_copy(row_hbm.at[pl.ds(ebase + s * SLABP, SLABP)], rin)
        pltpu.sync_copy(col_hbm.at[pl.ds(ebase + s * SLABP, SLABP)], cin)
        for j in range(SLABP // 16):
            sl = pl.ds(j * 16, 16)
            r = rin[sl]
            rm = jnp.where(r >= half16, r + shift16, r)
            c = cin[sl]
            ml = c < half16
            mh = jnp.logical_not(ml)
            mli = ml.astype(jnp.int32)
            mhi = 1 - mli
            incl = plsc.cumsum(mli)
            inch = plsc.cumsum(mhi)
            posl = (incl - mli) + cl16
            posh = (inch - mhi) + ch16
            plsc.store_scatter(lr, [posl], rm, mask=ml)
            plsc.store_scatter(lc, [posl], c, mask=ml)
            plsc.store_scatter(hr, [posh], rm, mask=mh)
            plsc.store_scatter(hc, [posh], c - half16, mask=mh)
            npl = plsc.all_reduce_population_count(ml)
            cl16 = cl16 + npl
            ch16 = ch16 + (16 - npl)
        return (cl16, ch16)

    zero16 = jnp.zeros((16,), jnp.int32)
    cl16, ch16 = lax.fori_loop(0, NSLABP, slab_body, (zero16, zero16))

    pltpu.sync_copy(lr.at[pl.ds(0, WCAP)], brow_hbm.at[0, pl.ds(ebase, WCAP)])
    pltpu.sync_copy(lc.at[pl.ds(0, WCAP)], bcol_hbm.at[0, pl.ds(ebase, WCAP)])
    pltpu.sync_copy(hr.at[pl.ds(0, WCAP)], brow_hbm.at[1, pl.ds(ebase, WCAP)])
    pltpu.sync_copy(hc.at[pl.ds(0, WCAP)], bcol_hbm.at[1, pl.ds(ebase, WCAP)])
    cntb[0, pl.ds(0, 16)] = cl16
    cntb[1, pl.ds(0, 16)] = ch16
    pltpu.sync_copy(cntb.at[0], cnts_hbm.at[0, w])
    pltpu.sync_copy(cntb.at[1], cnts_hbm.at[1, w])


@functools.partial(
    pl.kernel,
    mesh=_mesh,
    out_type=jax.ShapeDtypeStruct((PADN, 16), jnp.float32),
    scratch_types=[
        pltpu.VMEM((NW, 16), jnp.int32),
        pltpu.VMEM((SLAB,), jnp.int32),
        pltpu.VMEM((SLAB, 16), jnp.float32),
        pltpu.VMEM((ROWS_PT, 16), jnp.float32),
        pltpu.SemaphoreType.DMA,
        pltpu.VMEM_SHARED((PAD_HALF, 16), jnp.float32),
    ],
    compiler_params=_sc_params,
)
def _deg_kernel(bcol_hbm, cnts_hbm, dinv_hbm, cntv, loc1, ones_v, dstage,
                ssem, accd):
    sc = lax.axis_index("c")
    t = lax.axis_index("s")

    def init_body(i, carry):
        ones_v[i, pl.ds(0, 16)] = jnp.ones((16,), jnp.float32)
        return carry

    lax.fori_loop(0, SLAB, init_body, 0)

    def zstage_body(i, carry):
        dstage[i, pl.ds(0, 16)] = _zeros16()
        return carry

    lax.fori_loop(0, ROWS_PT, zstage_body, 0)
    pltpu.sync_copy(dstage, accd.at[pl.ds(t * ROWS_PT, ROWS_PT)])
    pltpu.sync_copy(cnts_hbm.at[sc], cntv)
    plsc.subcore_barrier()

    half16 = jnp.full((16,), HALF, jnp.int32)
    lane = lax.iota(jnp.int32, 16)

    for w in (t, t + NTILES):
        n = cntv[w, pl.ds(0, 16)][0]
        ns = (n + SLAB - 1) // SLAB
        woff = w * WCAP

        def slab_body(s, carry):
            pltpu.sync_copy(bcol_hbm.at[sc, pl.ds(woff + s * SLAB, SLAB)],
                            loc1)
            n16 = jnp.full((16,), n, jnp.int32)
            for j in range(SLAB // 16):
                sl = pl.ds(j * 16, 16)
                p16 = lane + (s * SLAB + j * 16)
                loc1[sl] = jnp.where(p16 < n16, loc1[sl], half16)
            pltpu.async_copy(ones_v, accd.at[loc1], ssem, add=True).wait()
            return carry

        lax.fori_loop(0, ns, slab_body, 0)

    plsc.subcore_barrier()

    pltpu.sync_copy(accd.at[pl.ds(t * ROWS_PT, ROWS_PT)], dstage)
    onef = jnp.ones((16,), jnp.float32)
    zerof = _zeros16()

    def inv_body(i, carry):
        d = dstage[i, pl.ds(0, 16)]
        dstage[i, pl.ds(0, 16)] = jnp.where(d > zerof, onef / d, zerof)
        return carry

    lax.fori_loop(0, ROWS_PT, inv_body, 0)
    pltpu.sync_copy(dstage, dinv_hbm.at[pl.ds(sc * PAD_HALF + t * ROWS_PT, ROWS_PT)])


@functools.partial(
    pl.kernel,
    mesh=_mesh,
    out_type=jax.ShapeDtypeStruct((PADN, D), jnp.float32),
    scratch_types=[
        pltpu.VMEM((NW, 16), jnp.int32),
        pltpu.VMEM((SLAB,), jnp.int32),
        pltpu.VMEM((SLAB,), jnp.int32),
        pltpu.VMEM((CPS, CH), jnp.int32),
        pltpu.VMEM((CPS, CH), jnp.int32),
        [pltpu.VMEM((CH, D), jnp.float32) for _ in range(NBUF)],
        pltpu.VMEM((BLK, 16), jnp.float32),
        pltpu.SemaphoreType.DMA,
        pltpu.SemaphoreType.DMA,
        pltpu.SemaphoreType.DMA,
        pltpu.VMEM_SHARED((PAD_HALF, D), jnp.float32),
    ],
    compiler_params=_sc_params,
)
def _layer_kernel(brow_hbm, bcol_hbm, cnts_hbm, dinv_hbm, x_hbm, out_hbm,
                  cntv, rowm1, loc1, rowm2, loc2, rbufs, dbuf, lsem, gsem,
                  ssem, acc):
    sc = lax.axis_index("c")
    t = lax.axis_index("s")
    buf = rbufs[0]

    def zbuf_body(i, carry):
        for j in range(D // 16):
            buf[i, pl.ds(j * 16, 16)] = _zeros16()
        return carry

    lax.fori_loop(0, BLK, zbuf_body, 0)

    def zacc_body(b, carry):
        pltpu.sync_copy(buf.at[pl.ds(0, BLK)],
                        acc.at[pl.ds(t * ROWS_PT + b * BLK, BLK)])
        return carry

    lax.fori_loop(0, NBLK, zacc_body, 0)
    pltpu.sync_copy(cnts_hbm.at[sc], cntv)
    plsc.subcore_barrier()

    half16 = jnp.full((16,), HALF, jnp.int32)
    zero16 = jnp.zeros((16,), jnp.int32)
    lane = lax.iota(jnp.int32, 16)
    nsub = CH // 16

    for w in (t, t + NTILES):
        n = cntv[w, pl.ds(0, 16)][0]
        ns = (n + SLAB - 1) // SLAB
        woff = w * WCAP

        def slab_body(s, carry):
            @pl.when(s * SLAB < n)
            def _():
                _do_slab(s)
            return carry

        def _do_slab(s):
            h1 = pltpu.async_copy(brow_hbm.at[sc, pl.ds(woff + s * SLAB, SLAB)],
                                  rowm1, lsem)
            h2 = pltpu.async_copy(bcol_hbm.at[sc, pl.ds(woff + s * SLAB, SLAB)],
                                  loc1, lsem)
            h1.wait()
            h2.wait()
            n16 = jnp.full((16,), n, jnp.int32)
            for j in range(SLAB // 16):
                sl = pl.ds(j * 16, 16)
                dst = (j // nsub, pl.ds((j % nsub) * 16, 16))
                p16 = lane + (s * SLAB + j * 16)
                ok = p16 < n16
                rowm2[dst] = jnp.where(ok, rowm1[sl], zero16)
                loc2[dst] = jnp.where(ok, loc1[sl], half16)
            ghs = [pltpu.async_copy(x_hbm.at[rowm2.at[k]],
                                    rbufs[k], gsem)
                   for k in range(NBUF)]
            shs = []
            for k in range(CPS):
                ghs[k].wait()
                shs.append(pltpu.async_copy(rbufs[k % NBUF],
                                            acc.at[loc2.at[k]], ssem,
                                            add=True))
                if k + NBUF < CPS:
                    shs[k].wait()
                    ghs.append(pltpu.async_copy(
                        x_hbm.at[rowm2.at[k + NBUF]],
                        rbufs[k % NBUF], gsem))
            for h in shs[CPS - NBUF:]:
                h.wait()

        lax.fori_loop(0, (WCAP + SLAB - 1) // SLAB, slab_body, 0)

    plsc.subcore_barrier()

    def scale_blk(b, carry):
        roff = t * ROWS_PT + b * BLK
        pltpu.sync_copy(acc.at[pl.ds(roff, BLK)], buf.at[pl.ds(0, BLK)])
        pltpu.sync_copy(dinv_hbm.at[pl.ds(sc * PAD_HALF + roff, BLK)], dbuf)

        def scale_row(r, c2):
            dv = dbuf[r, pl.ds(0, 16)]
            for j in range(D // 16):
                sl = pl.ds(j * 16, 16)
                buf[r, sl] = buf[r, sl] * dv
            return c2

        lax.fori_loop(0, BLK, scale_row, 0)
        pltpu.sync_copy(buf.at[pl.ds(0, BLK)],
                        out_hbm.at[pl.ds(sc * PAD_HALF + roff, BLK)])
        return carry

    lax.fori_loop(0, NBLK, scale_blk, 0)


def _mean_body(a, b, c, d, o):
    o[...] = (a[...] + b[...] + c[...] + d[...]) * 0.25


def _mean4(x0, x1, x2, x3):
    n = PADN * D // 128
    blk = (n // 16, 128)
    spec = pl.BlockSpec(blk, lambda i: (i, 0))
    f = pl.pallas_call(
        _mean_body,
        grid=(16,),
        in_specs=[spec] * 4,
        out_specs=spec,
        out_shape=jax.ShapeDtypeStruct((n, 128), jnp.float32),
    )
    r = lambda x: x.reshape(n, 128)
    return f(r(x0), r(x1), r(x2), r(x3)).reshape(PADN, D)


@jax.jit
def kernel(edge_index, user_emb, item_emb):
    row = edge_index[0]
    col = edge_index[1]
    rowp = jnp.pad(row, (0, EPAD - E))
    colp = jnp.pad(col, (0, EPAD - E), constant_values=N_NODES)

    x0 = jnp.zeros((PADN, D), jnp.float32)
    x0 = x0.at[0:HALF].set(user_emb)
    x0 = x0.at[PAD_HALF:PAD_HALF + HALF].set(item_emb)

    brow, bcol, cnts = _part_kernel(rowp, colp)
    dinv = _deg_kernel(bcol, cnts)
    x1 = _layer_kernel(brow, bcol, cnts, dinv, x0)
    x2 = _layer_kernel(brow, bcol, cnts, dinv, x1)
    x3 = _layer_kernel(brow, bcol, cnts, dinv, x2)

    m = _mean4(x0, x1, x2, x3)
    return m[0:HALF], m[PAD_HALF:PAD_HALF + HALF]


# speedup vs baseline: 1.4973x; 1.4973x over previous
import functools

import jax
import jax.numpy as jnp
from jax import lax
from jax.experimental import pallas as pl
from jax.experimental.pallas import tpu as pltpu
from jax.experimental.pallas import tpu_sc as plsc

N_USERS = 25000
N_ITEMS = 25000
HALF = 25000
N_NODES = N_USERS + N_ITEMS
D = 64
NUM_LAYERS = 3
E = 800000

NTILES = 16
ROWS_PT = 1568
PAD_HALF = NTILES * ROWS_PT
PADN = 2 * PAD_HALF
PAD_OFF = PAD_HALF - HALF

CH = 128
CPS = 8
SLAB = CH * CPS
NSLAB = 49
PER_TILE_E = NSLAB * SLAB
EPAD = NTILES * PER_TILE_E
EROWS_PT = PER_TILE_E // CH

NBUF = 3

BLK = 112
NBLK = ROWS_PT // BLK

_mesh = plsc.VectorSubcoreMesh(core_axis_name="c", subcore_axis_name="s")
_sc_params = pltpu.CompilerParams(use_tc_tiling_on_sc=False)


def _zeros16():
    return jnp.zeros((16,), jnp.float32)


@functools.partial(
    pl.kernel,
    mesh=_mesh,
    out_type=jax.ShapeDtypeStruct((PADN, 16), jnp.float32),
    scratch_types=[
        pltpu.VMEM((SLAB,), jnp.int32),
        pltpu.VMEM((SLAB, 16), jnp.float32),
        pltpu.VMEM((ROWS_PT, 16), jnp.float32),
        pltpu.SemaphoreType.DMA,
        pltpu.VMEM_SHARED((PAD_HALF, 16), jnp.float32),
    ],
    compiler_params=_sc_params,
)
def _deg_kernel(col_hbm, dinv_hbm, loc1, ones_v, dstage, ssem, accd):
    sc = lax.axis_index("c")
    t = lax.axis_index("s")
    base_dst = sc * HALF

    def init_body(i, carry):
        ones_v[i, pl.ds(0, 16)] = jnp.ones((16,), jnp.float32)
        return carry

    lax.fori_loop(0, SLAB, init_body, 0)

    def zstage_body(i, carry):
        dstage[i, pl.ds(0, 16)] = _zeros16()
        return carry

    lax.fori_loop(0, ROWS_PT, zstage_body, 0)
    pltpu.sync_copy(dstage, accd.at[pl.ds(t * ROWS_PT, ROWS_PT)])
    plsc.subcore_barrier()

    half16 = jnp.full((16,), HALF, jnp.int32)
    zero16 = jnp.zeros((16,), jnp.int32)

    def slab_body(s, carry):
        pltpu.sync_copy(col_hbm.at[pl.ds(t * PER_TILE_E + s * SLAB, SLAB)], loc1)
        for j in range(SLAB // 16):
            sl = pl.ds(j * 16, 16)
            local = loc1[sl] - base_dst
            ok = (local >= zero16) & (local < half16)
            loc1[sl] = jnp.where(ok, local, half16)
        pltpu.async_copy(ones_v, accd.at[loc1], ssem, add=True).wait()
        return carry

    lax.fori_loop(0, NSLAB, slab_body, 0)
    plsc.subcore_barrier()

    pltpu.sync_copy(accd.at[pl.ds(t * ROWS_PT, ROWS_PT)], dstage)
    onef = jnp.ones((16,), jnp.float32)
    zerof = _zeros16()

    def inv_body(i, carry):
        d = dstage[i, pl.ds(0, 16)]
        dstage[i, pl.ds(0, 16)] = jnp.where(d > zerof, onef / d, zerof)
        return carry

    lax.fori_loop(0, ROWS_PT, inv_body, 0)
    pltpu.sync_copy(dstage, dinv_hbm.at[pl.ds(sc * PAD_HALF + t * ROWS_PT, ROWS_PT)])


@functools.partial(
    pl.kernel,
    mesh=_mesh,
    out_type=jax.ShapeDtypeStruct((PADN, D), jnp.float32),
    scratch_types=[
        pltpu.VMEM((CPS, CH), jnp.int32),
        pltpu.VMEM((CPS, CH), jnp.int32),
        [pltpu.VMEM((CH, D), jnp.float32) for _ in range(NBUF)],
        pltpu.VMEM((BLK, 16), jnp.float32),
        pltpu.SemaphoreType.DMA,
        pltpu.SemaphoreType.DMA,
        pltpu.VMEM_SHARED((PAD_HALF, D), jnp.float32),
    ],
    compiler_params=_sc_params,
)
def _layer_kernel(row_hbm, col_hbm, dinv_hbm, x_hbm, out_hbm,
                  rowm2, loc2, rbufs, dbuf, gsem, ssem, acc):
    sc = lax.axis_index("c")
    t = lax.axis_index("s")
    base_dst = sc * HALF
    buf = rbufs[0]

    def zbuf_body(i, carry):
        for j in range(D // 16):
            buf[i, pl.ds(j * 16, 16)] = _zeros16()
        return carry

    lax.fori_loop(0, BLK, zbuf_body, 0)

    def zacc_body(b, carry):
        pltpu.sync_copy(buf.at[pl.ds(0, BLK)],
                        acc.at[pl.ds(t * ROWS_PT + b * BLK, BLK)])
        return carry

    lax.fori_loop(0, NBLK, zacc_body, 0)
    plsc.subcore_barrier()

    erow0 = t * EROWS_PT
    half16 = jnp.full((16,), HALF, jnp.int32)
    zero16 = jnp.zeros((16,), jnp.int32)
    shift16 = jnp.full((16,), PAD_OFF, jnp.int32)
    nsub = CH // 16

    def slab_body(s, carry):
        pltpu.sync_copy(row_hbm.at[pl.ds(erow0 + s * CPS, CPS)], rowm2)
        pltpu.sync_copy(col_hbm.at[pl.ds(erow0 + s * CPS, CPS)], loc2)
        for j in range(SLAB // 16):
            dst = (j // nsub, pl.ds((j % nsub) * 16, 16))
            r = rowm2[dst]
            rowm2[dst] = jnp.where(r >= half16, r + shift16, r)
            local = loc2[dst] - base_dst
            ok = (local >= zero16) & (local < half16)
            loc2[dst] = jnp.where(ok, local, half16)
        ghs = [pltpu.async_copy(x_hbm.at[rowm2.at[k]], rbufs[k], gsem)
               for k in range(NBUF)]
        shs = []
        for k in range(CPS):
            ghs[k].wait()
            shs.append(pltpu.async_copy(rbufs[k % NBUF], acc.at[loc2.at[k]],
                                        ssem, add=True))
            if k + NBUF < CPS:
                shs[k].wait()
                ghs.append(pltpu.async_copy(x_hbm.at[rowm2.at[k + NBUF]],
                                            rbufs[k % NBUF], gsem))
        for h in shs[CPS - NBUF:]:
            h.wait()
        return carry

    lax.fori_loop(0, NSLAB, slab_body, 0)
    plsc.subcore_barrier()

    def scale_blk(b, carry):
        roff = t * ROWS_PT + b * BLK
        pltpu.sync_copy(acc.at[pl.ds(roff, BLK)], buf.at[pl.ds(0, BLK)])
        pltpu.sync_copy(dinv_hbm.at[pl.ds(sc * PAD_HALF + roff, BLK)], dbuf)

        def scale_row(r, c2):
            dv = dbuf[r, pl.ds(0, 16)]
            for j in range(D // 16):
                sl = pl.ds(j * 16, 16)
                buf[r, sl] = buf[r, sl] * dv
            return c2

        lax.fori_loop(0, BLK, scale_row, 0)
        pltpu.sync_copy(buf.at[pl.ds(0, BLK)],
                        out_hbm.at[pl.ds(sc * PAD_HALF + roff, BLK)])
        return carry

    lax.fori_loop(0, NBLK, scale_blk, 0)


def _mean_body(a, b, c, d, o):
    o[...] = (a[...] + b[...] + c[...] + d[...]) * 0.25


def _mean4(x0, x1, x2, x3):
    n = PADN * D // 128
    blk = (n // 16, 128)
    spec = pl.BlockSpec(blk, lambda i: (i, 0))
    f = pl.pallas_call(
        _mean_body,
        grid=(16,),
        in_specs=[spec] * 4,
        out_specs=spec,
        out_shape=jax.ShapeDtypeStruct((n, 128), jnp.float32),
    )
    r = lambda x: x.reshape(n, 128)
    return f(r(x0), r(x1), r(x2), r(x3)).reshape(PADN, D)


@jax.jit
def kernel(edge_index, user_emb, item_emb):
    row = edge_index[0]
    col = edge_index[1]
    rowp = jnp.pad(row, (0, EPAD - E)).reshape(EPAD // CH, CH)
    colp1 = jnp.pad(col, (0, EPAD - E), constant_values=N_NODES)
    colp = colp1.reshape(EPAD // CH, CH)

    x0 = jnp.zeros((PADN, D), jnp.float32)
    x0 = x0.at[0:HALF].set(user_emb)
    x0 = x0.at[PAD_HALF:PAD_HALF + HALF].set(item_emb)

    dinv = _deg_kernel(colp1)
    x1 = _layer_kernel(rowp, colp, dinv, x0)
    x2 = _layer_kernel(rowp, colp, dinv, x1)
    x3 = _layer_kernel(rowp, colp, dinv, x2)

    m = _mean4(x0, x1, x2, x3)
    return m[0:HALF], m[PAD_HALF:PAD_HALF + HALF]
